# baseline (speedup 1.0000x reference)
import jax
import jax.numpy as jnp
from jax import lax
from jax.experimental import pallas as pl
from jax.experimental.pallas import tpu as pltpu
from jax.experimental.pallas import tpu_sc as plsc

PAD = 1
B = 4
T = 2048
D = 1024
NC = 2
NS = 16
L = 16
NW = NC * NS
TOK_PER_W = (B * T) // NW
SEG_PER_ROW = T // TOK_PER_W
CHUNK = 32
NCHUNK = TOK_PER_W // CHUNK
VREGS_PER_SEG = TOK_PER_W // L
VPC = CHUNK // L
NBUF = 3


def _body(x_hbm, table_hbm, out_hbm, xrow_ref, *rest):
    idxs = rest[:NCHUNK]
    bufs = rest[NCHUNK:NCHUNK + NBUF]
    gsems = rest[NCHUNK + NBUF:NCHUNK + 2 * NBUF]
    ssems = rest[NCHUNK + 2 * NBUF:NCHUNK + 3 * NBUF]

    wid = lax.axis_index("s") * NC + lax.axis_index("c")
    row = wid // SEG_PER_ROW
    seg = wid % SEG_PER_ROW

    pltpu.sync_copy(x_hbm.at[row], xrow_ref)

    def acc_body(j, acc_v):
        v = xrow_ref[pl.ds(j * L, L)]
        return acc_v + (v != PAD).astype(jnp.int32)

    acc_v = lax.fori_loop(
        0, seg * VREGS_PER_SEG, acc_body, jnp.zeros((L,), jnp.int32)
    )
    carry = jnp.sum(acc_v)

    base = wid * TOK_PER_W
    handles_g = [None] * NBUF
    handles_s = [None] * NBUF
    for c in range(NCHUNK):
        for kk in range(VPC):
            k = c * VPC + kk
            i = seg * VREGS_PER_SEG + k
            v = xrow_ref[pl.ds(i * L, L)]
            m = (v != PAD).astype(jnp.int32)
            pos = (jnp.cumsum(m) + carry) * m + PAD
            idxs[c][pl.ds(kk * L, L)] = pos
            carry = carry + jnp.sum(m)
        b = c % NBUF
        if handles_s[b] is not None:
            handles_s[b].wait()
        handles_g[b] = pltpu.async_copy(table_hbm.at[idxs[c]], bufs[b], gsems[b])
        d = c - (NBUF - 1)
        if d >= 0:
            db = d % NBUF
            handles_g[db].wait()
            handles_s[db] = pltpu.async_copy(
                bufs[db], out_hbm.at[pl.ds(base + d * CHUNK, CHUNK)], ssems[db]
            )
    for d in range(max(0, NCHUNK - NBUF + 1), NCHUNK):
        db = d % NBUF
        handles_g[db].wait()
        handles_s[db] = pltpu.async_copy(
            bufs[db], out_hbm.at[pl.ds(base + d * CHUNK, CHUNK)], ssems[db]
        )
    for b in range(NBUF):
        if handles_s[b] is not None:
            handles_s[b].wait()


_lookup = pl.kernel(
    _body,
    out_type=jax.ShapeDtypeStruct((B * T, D), jnp.float32),
    mesh=plsc.VectorSubcoreMesh(
        core_axis_name="c", subcore_axis_name="s", num_cores=NC, num_subcores=NS
    ),
    scratch_types=(
        [pltpu.VMEM((T,), jnp.int32)]
        + [pltpu.VMEM((CHUNK,), jnp.int32) for _ in range(NCHUNK)]
        + [pltpu.VMEM((CHUNK, D), jnp.float32) for _ in range(NBUF)]
        + [pltpu.SemaphoreType.DMA for _ in range(2 * NBUF)]
    ),
    compiler_params=pltpu.CompilerParams(needs_layout_passes=False),
)


def kernel(x, table):
    out = _lookup(x, table)
    return out.reshape(B, T, D)
